# baseline (speedup 1.0000x reference)
import functools

import jax
import jax.numpy as jnp
from jax import lax
from jax.experimental import pallas as pl
from jax.experimental.pallas import tpu as pltpu
from jax.experimental.pallas import tpu_sc as plsc

N = 10000
E = 320000
D = 128
B = 4
L = 512
C = 1000
T = 12

NC = 2
NS = 16
LN = 16


_EW = E // NS
_KC = 80
_NSTG = 25
_NBLK = _EW // (_KC * _NSTG)
_ZR = 128


def _sc_aggregate(h_all, src3d, dst3d, norm3d):
    mesh = plsc.VectorSubcoreMesh(core_axis_name="c", subcore_axis_name="s",
                                  num_cores=NC, num_subcores=NS)

    @functools.partial(
        pl.kernel, mesh=mesh,
        out_type=jax.ShapeDtypeStruct((NC, N, D), jnp.float32),
        scratch_types=[
            pltpu.VMEM((_NSTG, _KC), jnp.int32),
            pltpu.VMEM((_NSTG, _KC), jnp.int32),
            pltpu.VMEM((_NSTG, _KC), jnp.float32),
            pltpu.VMEM((_KC, D), jnp.float32),
            pltpu.VMEM((_ZR, D), jnp.float32),
            pltpu.VMEM_SHARED((N, D), jnp.float32),
            pltpu.SemaphoreType.DMA,
        ],
    )
    def agg(h_hbm, src_hbm, dst_hbm, norm_hbm, out_hbm,
            src_v, dst_v, norm_v, rows_v, zero_v, acc_sh, sem):
        c = lax.axis_index("c")
        s = lax.axis_index("s")

        zeros16 = jnp.zeros((LN,), jnp.float32)

        def zrow(i, _):
            for m in range(D // LN):
                zero_v[i, pl.ds(m * LN, LN)] = zeros16
            return 0
        lax.fori_loop(0, _ZR, zrow, 0)
        @pl.when(s < NS - 1)
        def _():
            for k in range(5):
                pltpu.sync_copy(zero_v,
                                acc_sh.at[pl.ds(s * 640 + k * _ZR, _ZR)])

        @pl.when(s == NS - 1)
        def _():
            for k in range(3):
                pltpu.sync_copy(zero_v,
                                acc_sh.at[pl.ds(9600 + k * _ZR, _ZR)])
            pltpu.sync_copy(zero_v.at[pl.ds(0, 16)],
                            acc_sh.at[pl.ds(9984, 16)])
        plsc.subcore_barrier()

        blk0 = c * (NS * _NBLK) + s * _NBLK

        def blk(b, _):
            r = blk0 + b
            pltpu.sync_copy(src_hbm.at[r], src_v)
            pltpu.sync_copy(dst_hbm.at[r], dst_v)
            pltpu.sync_copy(norm_hbm.at[r], norm_v)

            def chunk(j, _):
                pltpu.async_copy(h_hbm.at[src_v.at[j]], rows_v, sem).wait()

                def grp(g, _):
                    base = g * LN
                    norm16 = norm_v[j, pl.ds(base, LN)]
                    for k in range(LN):
                        sc = norm16[k]
                        for m in range(D // LN):
                            sl = pl.ds(m * LN, LN)
                            rows_v[base + k, sl] = rows_v[base + k, sl] * sc
                    return 0
                lax.fori_loop(0, _KC // LN, grp, 0)
                pltpu.sync_copy(rows_v, acc_sh.at[dst_v.at[j]], add=True)
                return 0
            lax.fori_loop(0, _NSTG, chunk, 0)
            return 0
        lax.fori_loop(0, _NBLK, blk, 0)

        plsc.subcore_barrier()
        @pl.when(s < NS - 1)
        def _():
            for k in range(5):
                sl = pl.ds(s * 640 + k * _ZR, _ZR)
                pltpu.sync_copy(acc_sh.at[sl], out_hbm.at[c].at[sl])

        @pl.when(s == NS - 1)
        def _():
            for k in range(3):
                sl = pl.ds(9600 + k * _ZR, _ZR)
                pltpu.sync_copy(acc_sh.at[sl], out_hbm.at[c].at[sl])
            sl = pl.ds(9984, 16)
            pltpu.sync_copy(acc_sh.at[sl], out_hbm.at[c].at[sl])

    return agg(h_all, src3d, dst3d, norm3d)


def _sc_deg(dst3d, ew3d):
    mesh = plsc.VectorSubcoreMesh(core_axis_name="c", subcore_axis_name="s",
                                  num_cores=NC, num_subcores=NS)

    @functools.partial(
        pl.kernel, mesh=mesh,
        out_type=jax.ShapeDtypeStruct((NC, N, D), jnp.float32),
        scratch_types=[
            pltpu.VMEM((_NSTG, _KC), jnp.int32),
            pltpu.VMEM((_NSTG, _KC), jnp.float32),
            pltpu.VMEM((_KC, D), jnp.float32),
            pltpu.VMEM((_ZR, D), jnp.float32),
            pltpu.VMEM_SHARED((N, D), jnp.float32),
        ],
    )
    def deg(dst_hbm, ew_hbm, deg_hbm, dst_v, ew_v, upd_v, zero_v, deg_sh):
        c = lax.axis_index("c")
        s = lax.axis_index("s")
        zeros16 = jnp.zeros((LN,), jnp.float32)

        def zup(i, _):
            for m in range(D // LN):
                upd_v[i, pl.ds(m * LN, LN)] = zeros16
            return 0
        lax.fori_loop(0, _KC, zup, 0)

        def zrow(i, _):
            for m in range(D // LN):
                zero_v[i, pl.ds(m * LN, LN)] = zeros16
            return 0
        lax.fori_loop(0, _ZR, zrow, 0)

        @pl.when(s < NS - 1)
        def _():
            for k in range(5):
                pltpu.sync_copy(zero_v,
                                deg_sh.at[pl.ds(s * 640 + k * _ZR, _ZR)])

        @pl.when(s == NS - 1)
        def _():
            for k in range(3):
                pltpu.sync_copy(zero_v,
                                deg_sh.at[pl.ds(9600 + k * _ZR, _ZR)])
            pltpu.sync_copy(zero_v.at[pl.ds(0, 16)],
                            deg_sh.at[pl.ds(9984, 16)])
        plsc.subcore_barrier()

        blk0 = c * (NS * _NBLK) + s * _NBLK

        def blk1(b, _):
            r = blk0 + b
            pltpu.sync_copy(dst_hbm.at[r], dst_v)
            pltpu.sync_copy(ew_hbm.at[r], ew_v)

            def chunk(j, _):
                for g in range(_KC // LN):
                    ew16 = ew_v[j, pl.ds(g * LN, LN)]
                    for k in range(LN):
                        upd_v[g * LN + k, pl.ds(0, LN)] = jnp.full(
                            (LN,), ew16[k], jnp.float32)
                pltpu.sync_copy(upd_v, deg_sh.at[dst_v.at[j]], add=True)
                return 0
            lax.fori_loop(0, _NSTG, chunk, 0)
            return 0
        lax.fori_loop(0, _NBLK, blk1, 0)
        plsc.subcore_barrier()

        @pl.when(s < NS - 1)
        def _():
            for k in range(5):
                sl = pl.ds(s * 640 + k * _ZR, _ZR)
                pltpu.sync_copy(deg_sh.at[sl], deg_hbm.at[c].at[sl])

        @pl.when(s == NS - 1)
        def _():
            for k in range(3):
                sl = pl.ds(9600 + k * _ZR, _ZR)
                pltpu.sync_copy(deg_sh.at[sl], deg_hbm.at[c].at[sl])
            sl = pl.ds(9984, 16)
            pltpu.sync_copy(deg_sh.at[sl], deg_hbm.at[c].at[sl])

    return deg(dst3d, ew3d)


def _tc_dinv_body(deg_ref, c_ref):
    d = deg_ref[:, :, 0] + 1.0
    c_ref[...] = lax.rsqrt(d)


def _tc_dinv(deg16):
    return pl.pallas_call(
        _tc_dinv_body,
        out_shape=jax.ShapeDtypeStruct((NC, N), jnp.float32),
    )(deg16)


_RT = 1000


def _tc_proj_body(x_ref, w_ref, dinv_ref, h_ref, hs_ref):
    x = x_ref[...]
    w = w_ref[0]
    h = lax.dot_general(x, w, (((1,), (1,)), ((), ())),
                        preferred_element_type=jnp.float32)
    h_ref[0] = h
    hs_ref[0] = h * dinv_ref[0]


def _tc_proj(x0, w2, dinv):
    return pl.pallas_call(
        _tc_proj_body,
        grid=(NC, N // _RT),
        in_specs=[
            pl.BlockSpec((_RT, D), lambda g, t: (t, 0)),
            pl.BlockSpec((1, D, D), lambda g, t: (g, 0, 0)),
            pl.BlockSpec((1, _RT, 1), lambda g, t: (g, t, 0)),
        ],
        out_specs=[
            pl.BlockSpec((1, _RT, D), lambda g, t: (g, t, 0)),
            pl.BlockSpec((1, _RT, D), lambda g, t: (g, t, 0)),
        ],
        out_shape=[jax.ShapeDtypeStruct((NC, N, D), jnp.float32),
                   jax.ShapeDtypeStruct((NC, N, D), jnp.float32)],
    )(x0, w2, dinv)


def _tc_epi_body(agg_ref, h_ref, dinv_ref, b_ref, w_ref,
                 x_ref, hn_ref, hsn_ref):
    dinv = dinv_ref[0]
    x = jax.nn.relu(dinv * agg_ref[0] + (dinv * dinv) * h_ref[0] + b_ref[0])
    x_ref[0] = x
    w = w_ref[0]
    hn = lax.dot_general(x, w, (((1,), (1,)), ((), ())),
                         preferred_element_type=jnp.float32)
    hn_ref[0] = hn
    hsn_ref[0] = hn * dinv


def _tc_epi_proj(agg, h, dinv, b_l, w2):
    return pl.pallas_call(
        _tc_epi_body,
        grid=(NC, N // _RT),
        in_specs=[
            pl.BlockSpec((1, _RT, D), lambda g, t: (g, t, 0)),
            pl.BlockSpec((1, _RT, D), lambda g, t: (g, t, 0)),
            pl.BlockSpec((1, _RT, 1), lambda g, t: (g, t, 0)),
            pl.BlockSpec((1, 1, D), lambda g, t: (g, 0, 0)),
            pl.BlockSpec((1, D, D), lambda g, t: (g, 0, 0)),
        ],
        out_specs=[
            pl.BlockSpec((1, _RT, D), lambda g, t: (g, t, 0)),
            pl.BlockSpec((1, _RT, D), lambda g, t: (g, t, 0)),
            pl.BlockSpec((1, _RT, D), lambda g, t: (g, t, 0)),
        ],
        out_shape=[jax.ShapeDtypeStruct((NC, N, D), jnp.float32)] * 3,
    )(agg, h, dinv, b_l, w2)


def _tc_epi_only_body(agg_ref, h_ref, dinv_ref, b_ref, x_ref):
    dinv = dinv_ref[0]
    x_ref[0] = jax.nn.relu(dinv * agg_ref[0] + (dinv * dinv) * h_ref[0]
                           + b_ref[0])


def _tc_epi_only(agg, h, dinv, b_l):
    return pl.pallas_call(
        _tc_epi_only_body,
        grid=(NC, N // _RT),
        in_specs=[
            pl.BlockSpec((1, _RT, D), lambda g, t: (g, t, 0)),
            pl.BlockSpec((1, _RT, D), lambda g, t: (g, t, 0)),
            pl.BlockSpec((1, _RT, 1), lambda g, t: (g, t, 0)),
            pl.BlockSpec((1, 1, D), lambda g, t: (g, 0, 0)),
        ],
        out_specs=pl.BlockSpec((1, _RT, D), lambda g, t: (g, t, 0)),
        out_shape=jax.ShapeDtypeStruct((NC, N, D), jnp.float32),
    )(agg, h, dinv, b_l)


def _tc_codebook_body(x_ref, w_ref, b_ref, o_ref):
    y = lax.dot_general(x_ref[...], w_ref[...], (((1,), (1,)), ((), ())),
                        preferred_element_type=jnp.float32)
    o_ref[...] = y + b_ref[...]


def _tc_codebook(word_embs, pnc_W, pnc_b):
    return pl.pallas_call(
        _tc_codebook_body,
        grid=(N // _RT,),
        in_specs=[
            pl.BlockSpec((_RT, D), lambda t: (t, 0)),
            pl.BlockSpec((D, D), lambda t: (0, 0)),
            pl.BlockSpec((1, D), lambda t: (0, 0)),
        ],
        out_specs=pl.BlockSpec((_RT, D), lambda t: (t, 0)),
        out_shape=jax.ShapeDtypeStruct((N, D), jnp.float32),
    )(word_embs, pnc_W, pnc_b[None, :])


_CP = 1008


def _tc_classmat_body(a_ref, b_ref, o_ref):
    @pl.when(pl.program_id(0) == 0)
    def _():
        o_ref[...] = jnp.zeros_like(o_ref)
    o_ref[...] += jnp.dot(a_ref[...], b_ref[...],
                          preferred_element_type=jnp.float32)


_KP = 10240


def _tc_classmat(graph_pad, codebook):
    return pl.pallas_call(
        _tc_classmat_body,
        grid=(_KP // 1024,),
        in_specs=[
            pl.BlockSpec((_CP, 1024), lambda t: (0, t)),
            pl.BlockSpec((1024, D), lambda t: (t, 0)),
        ],
        out_specs=pl.BlockSpec((_CP, D), lambda t: (0, 0)),
        out_shape=jax.ShapeDtypeStruct((_CP, D), jnp.float32),
    )(graph_pad, codebook)


_GR = 14336
_GW = _GR // (NC * NS)
_GCH = 64


def _sc_gather(table, idx3d):
    mesh = plsc.VectorSubcoreMesh(core_axis_name="c", subcore_axis_name="s",
                                  num_cores=NC, num_subcores=NS)

    @functools.partial(
        pl.kernel, mesh=mesh,
        out_type=jax.ShapeDtypeStruct((_GR, D), jnp.float32),
        scratch_types=[
            pltpu.VMEM((_GW // _GCH, _GCH), jnp.int32),
            pltpu.VMEM((_GCH, D), jnp.float32),
            pltpu.SemaphoreType.DMA,
        ],
    )
    def gat(tab_hbm, idx_hbm, out_hbm, idx_v, rows_v, sem):
        c = lax.axis_index("c")
        s = lax.axis_index("s")
        w = c * NS + s
        pltpu.sync_copy(idx_hbm.at[w], idx_v)
        for cc in range(_GW // _GCH):
            pltpu.async_copy(tab_hbm.at[idx_v.at[cc]], rows_v, sem).wait()
            pltpu.sync_copy(rows_v,
                            out_hbm.at[pl.ds(w * _GW + cc * _GCH, _GCH)])

    return gat(table, idx3d)


def _tc_mean3_body(g_ref, o_ref):
    i = pl.program_id(0)

    @pl.when(i < 2)
    def _():
        base = 3 * i
        acc = (g_ref[pl.ds(base, 1)][0] + g_ref[pl.ds(base + 1, 1)][0] +
               g_ref[pl.ds(base + 2, 1)][0])
        o_ref[0] = acc * (1.0 / 3.0)

    @pl.when(i == 2)
    def _():
        o_ref[0] = g_ref[pl.ds(6, 1)][0]


def _tc_mean3(gathered):
    return pl.pallas_call(
        _tc_mean3_body,
        grid=(3,),
        in_specs=[pl.BlockSpec((7, B * L, D), lambda i: (0, 0, 0))],
        out_specs=pl.BlockSpec((1, B * L, D), lambda i: (i, 0, 0)),
        out_shape=jax.ShapeDtypeStruct((3, B * L, D), jnp.float32),
    )(gathered)


def _assemble_body(x_ref, w_ref, b_ref, m_ref, o_ref):
    x = x_ref[0, 0]
    w = w_ref[0, 0]
    y = lax.dot_general(x, w, (((1,), (1,)), ((), ())),
                        preferred_element_type=jnp.float32)
    y = (y + b_ref[0, 0, 0]) * m_ref[0, 0]
    o_ref[0, 0, 0, 0:1, :] = jnp.zeros((1, D), jnp.float32)
    o_ref[0, 0, 0, 1:, :] = y


def _assemble(embs_all, w_all, b_all, m_all):
    grid = (3, T, B)
    return pl.pallas_call(
        _assemble_body,
        grid=grid,
        in_specs=[
            pl.BlockSpec((1, 1, L, D), lambda i, t, b: (i, b, 0, 0)),
            pl.BlockSpec((1, 1, D, D), lambda i, t, b: (i, t, 0, 0)),
            pl.BlockSpec((1, 1, 1, D), lambda i, t, b: (i, t, 0, 0)),
            pl.BlockSpec((1, 1, L, 1), lambda i, t, b: (i, b, 0, 0)),
        ],
        out_specs=pl.BlockSpec((1, 1, 1, L + 1, D),
                               lambda i, t, b: (i, t, b, 0, 0)),
        out_shape=jax.ShapeDtypeStruct((3, T, B, L + 1, D), jnp.float32),
    )(embs_all, w_all, b_all, m_all)


def _gcn_both(input_ids, word_embs, co_edge_index, co_edge_weight,
              re_edge_index, re_edge_weight, co_W, co_b, re_W, re_b):
    co_src, co_dst = co_edge_index[0], co_edge_index[1]
    re_src, re_dst = re_edge_index[0], re_edge_index[1]
    srcl3d = jnp.concatenate([co_src, re_src]).reshape(-1, _NSTG, _KC)
    src3d = jnp.concatenate([co_src, re_src + N]).reshape(-1, _NSTG, _KC)
    dst3d = jnp.concatenate([co_dst, re_dst]).reshape(-1, _NSTG, _KC)
    ew3d = jnp.concatenate([co_edge_weight,
                            re_edge_weight]).reshape(-1, _NSTG, _KC)
    deg16 = _sc_deg(dst3d, ew3d)
    dinv = _tc_dinv(deg16)[:, :, None]
    W2 = jnp.stack([co_W, re_W])
    b2 = jnp.stack([co_b, re_b])

    h, hs = _tc_proj(word_embs, W2[:, 0], dinv)
    xs = []
    for layer in range(3):
        agg = _sc_aggregate(hs.reshape(2 * N, D), src3d, dst3d, ew3d)
        b_l = b2[:, layer][:, None, :]
        if layer < 2:
            x, hn, hsn = _tc_epi_proj(agg, h, dinv, b_l, W2[:, layer + 1])
            h, hs = hn, hsn
        else:
            x = _tc_epi_only(agg, h, dinv, b_l)
        xs.append(x)
    return jnp.stack(xs, axis=0)


def kernel(word_embs, input_ids, class_ids, mask, co_edge_index,
           co_edge_weight, re_edge_index, re_edge_weight, pnc_graph,
           co_W, co_b, re_W, re_b, pnc_W, pnc_b, zc_W, zc_b, zcp_W, zcp_b):
    xs = _gcn_both(input_ids, word_embs, co_edge_index,
                   co_edge_weight, re_edge_index,
                   re_edge_weight, co_W, co_b, re_W, re_b)

    codebook_conv = _tc_codebook(word_embs, pnc_W, pnc_b)
    graph_pad = jnp.zeros((_CP, _KP), jnp.float32).at[:C + 1, :N].set(
        pnc_graph)
    cb_pad = jnp.zeros((_KP, D), jnp.float32).at[:N].set(codebook_conv)
    classmat = _tc_classmat(graph_pad, cb_pad)

    NP = N + 8
    xs_pad = jnp.zeros((2, 3, NP, D), jnp.float32).at[:, :, :N, :].set(
        jnp.transpose(xs, (1, 0, 2, 3)))
    table = jnp.concatenate([xs_pad.reshape(6 * NP, D),
                             classmat], axis=0)
    idx_flat = input_ids.reshape(-1)
    layer_idx = (jnp.arange(6, dtype=jnp.int32)[:, None] * NP + idx_flat)
    cls_idx = (6 * NP + class_ids.reshape(-1))[None, :]
    idx3d = jnp.concatenate([layer_idx, cls_idx],
                            axis=0).reshape(NC * NS, _GW // _GCH, _GCH)
    gathered = _sc_gather(table, idx3d).reshape(7, B * L, D)
    embs3 = _tc_mean3(gathered)

    co_re_mask = (~mask).astype(jnp.float32)
    embs_all = embs3.reshape(3, B, L, D)
    w_all = jnp.stack([zc_W, zc_W, zcp_W], axis=0)
    b_all = jnp.stack([zc_b, zc_b, zcp_b], axis=0)[:, :, None, :]
    m_all = jnp.stack([co_re_mask, co_re_mask,
                       jnp.ones_like(co_re_mask)], axis=0)

    return _assemble(embs_all, w_all, b_all, m_all)
